# baseline (speedup 1.0000x reference)
import functools

import jax
import jax.numpy as jnp
from jax import lax
from jax.experimental import pallas as pl
from jax.experimental.pallas import tpu as pltpu
from jax.experimental.pallas import tpu_sc as plsc

_K = 13
_EPS = 1e-9
_B, _N, _L = 8, 64, 8400
_NCH = _L // 16
_NC, _NS = 2, 16
_NW = _NC * _NS
_TRS = _B * _N // 8
_TPW = _TRS // _NW
_G = 8
_CAP = 1008
_CSTR = 1024
_HW = 4224
_NEG = -3.0e38


def _sc_body(m_hbm, o_hbm, inb, candi, selbuf, dsem):
    wid = lax.axis_index("s") * _NC + lax.axis_index("c")
    iota = lax.iota(jnp.int32, 16)
    onesv = jnp.full((16,), 1.0, jnp.float32)
    zerosv = jnp.zeros((16,), jnp.float32)
    zerosiv = jnp.zeros((16,), jnp.int32)

    def tile_row(t, carry):
        pltpu.async_copy(m_hbm.at[pl.ds(8 * t, 8), :], inb, dsem).wait()

        @plsc.parallel_loop(
            0, _NCH, unroll=2,
            carry=tuple(jnp.full((16,), _NEG, jnp.float32)
                        for _ in range(_G)))
        def accs(i, acc):
            sl = pl.ds(i * 16, 16)
            return tuple(jnp.maximum(acc[g], inb[g, sl]) for g in range(_G))

        b0v = []
        for g in range(_G):
            sk, _ = plsc.sort_key_val(accs[g], iota, descending=True)
            b0v.append(jnp.full((16,), sk[_K - 1]))

        @plsc.parallel_loop(
            0, _NCH, unroll=2,
            carry=tuple(jnp.zeros((), jnp.int32) for _ in range(_G)))
        def offs(i, off):
            sl = pl.ds(i * 16, 16)
            idxv = iota + i * 16
            new = []
            for g in range(_G):
                v = inb[g, sl]
                msk = v >= b0v[g]
                base = g * _CSTR + jnp.minimum(off[g], _CAP)
                plsc.store_compressed(candi.at[pl.ds(base, 16)], idxv,
                                      mask=msk)
                new.append(off[g]
                           + plsc.all_reduce_population_count(msk)[0])
            return tuple(new)

        for g in range(_G):
            selbuf[pl.ds(g * 16, 16)] = jnp.full((16,), -1, jnp.int32)
            ncand = offs[g]
            cvec = jnp.full((16,), ncand, jnp.int32)

            def merge_step(vals, idxs, run):
                sa, _ = plsc.sort_key_val(vals, idxs, descending=False)
                mx = jnp.maximum(run, sa)
                rd, _ = plsc.sort_key_val(mx, iota, descending=True)
                return rd

            def select_plan(run):
                thr = run[_K - 1]
                rowmax = run[0]
                cg = jnp.sum((run > thr).astype(jnp.int32))
                ne = jnp.full((16,), _K - cg, jnp.int32)
                return thr, rowmax, ne

            def select_step(vals, idxs, thrv, nev, eqc, soff):
                gt = vals > thrv
                eq = vals == thrv
                eqi = eq.astype(jnp.int32)
                excl = plsc.cumsum(eqi) - eqi
                sel = jnp.logical_or(
                    gt, jnp.logical_and(eq, (excl + eqc) < nev))
                seli = sel.astype(jnp.int32)
                sx = plsc.cumsum(seli) - seli
                plsc.store_scatter(selbuf, [soff + sx], idxs, mask=sel)
                return (eqc + plsc.all_reduce_population_count(eq),
                        soff + plsc.all_reduce_population_count(sel))

            @pl.when(ncand <= _CAP)
            def _(g=g, ncand=ncand, cvec=cvec):
                base = g * _CSTR
                candi[pl.ds(base + ncand, 16)] = zerosiv
                nch = (ncand + 15) // 16

                def body_c(j, run):
                    idxs = candi[pl.ds(base + j * 16, 16)]
                    vals = plsc.load_gather(inb, [jnp.full((16,), g, jnp.int32), idxs])
                    vals = jnp.where(iota + j * 16 < cvec, vals, -1.0)
                    return merge_step(vals, idxs, run)
                run = lax.fori_loop(0, nch, body_c,
                                    jnp.full((16,), _NEG, jnp.float32))
                thr, rowmax, ne = select_plan(run)
                thrv = jnp.full((16,), thr)

                @pl.when(rowmax > _EPS)
                def _():
                    def body_s(j, c):
                        idxs = candi[pl.ds(base + j * 16, 16)]
                        vals = plsc.load_gather(inb, [jnp.full((16,), g, jnp.int32), idxs])
                        vals = jnp.where(iota + j * 16 < cvec, vals, -1.0)
                        return select_step(vals, idxs, thrv, ne, *c)
                    lax.fori_loop(0, nch, body_s,
                                  (zerosiv, jnp.full((16,), g * 16,
                                                     jnp.int32)))

            @pl.when(ncand > _CAP)
            def _(g=g):
                def body_c(j, run):
                    sl = pl.ds(j * 16, 16)
                    return merge_step(inb[g, sl], iota + j * 16, run)
                run = lax.fori_loop(0, _NCH, body_c,
                                    jnp.full((16,), _NEG, jnp.float32))
                thr, rowmax, ne = select_plan(run)
                thrv = jnp.full((16,), thr)

                @pl.when(rowmax > _EPS)
                def _():
                    def body_s(j, c):
                        sl = pl.ds(j * 16, 16)
                        return select_step(inb[g, sl], iota + j * 16,
                                           thrv, ne, *c)
                    lax.fori_loop(0, _NCH, body_s,
                                  (zerosiv, jnp.full((16,), g * 16,
                                                     jnp.int32)))

        @plsc.parallel_loop(0, _NCH, unroll=2)
        def _(i):
            sl = pl.ds(i * 16, 16)
            for g in range(_G):
                inb[g, sl] = zerosv

        for g in range(_G):
            selv = selbuf[pl.ds(g * 16, 16)]
            m = selv >= 0
            gv = jnp.full((16,), g, jnp.int32)
            plsc.store_scatter(inb, [gv, selv], onesv, mask=m)
        pltpu.async_copy(inb, o_hbm.at[pl.ds(8 * t, 8), :], dsem).wait()
        return carry

    lax.fori_loop(0, _TPW, lambda k, c: tile_row(wid * _TPW + k, c),
                  jnp.zeros((), jnp.int32))


_mesh = plsc.VectorSubcoreMesh(core_axis_name="c", subcore_axis_name="s",
                               num_cores=_NC, num_subcores=_NS)

_topk_mask = functools.partial(
    pl.kernel,
    out_type=jax.ShapeDtypeStruct((_B * _N, _L), jnp.float32),
    mesh=_mesh,
    scratch_types=(
        pltpu.VMEM((_G, _L), jnp.float32),
        pltpu.VMEM((_G * _CSTR,), jnp.int32),
        pltpu.VMEM((_G * 16,), jnp.int32),
        pltpu.SemaphoreType.DMA,
    ),
    compiler_params=pltpu.CompilerParams(needs_layout_passes=False),
)(_sc_body)


@jax.jit
def kernel(metrics):
    b, n, l = metrics.shape
    out = _topk_mask(metrics.reshape(b * n, l))
    return out.reshape(b, n, l)


# speedup vs baseline: 1.0854x; 1.0854x over previous
import functools

import jax
import jax.numpy as jnp
from jax import lax
from jax.experimental import pallas as pl
from jax.experimental.pallas import tpu as pltpu
from jax.experimental.pallas import tpu_sc as plsc

_K = 13
_EPS = 1e-9
_B, _N, _L = 8, 64, 8400
_NCH = _L // 16
_NC, _NS = 2, 16
_NW = _NC * _NS
_TRS = _B * _N // 8
_TPW = _TRS // _NW
_G = 8
_CAP = 1008
_CSTR = 1024
_HW = 4224
_NEG = -3.0e38


def _sc_body(m_hbm, o_hbm, inb, candi, selbuf, dsem):
    wid = lax.axis_index("s") * _NC + lax.axis_index("c")
    iota = lax.iota(jnp.int32, 16)
    onesv = jnp.full((16,), 1.0, jnp.float32)
    zerosv = jnp.zeros((16,), jnp.float32)
    zerosiv = jnp.zeros((16,), jnp.int32)

    def tile_row(t, carry):
        pltpu.async_copy(m_hbm.at[pl.ds(8 * t, 8), :], inb, dsem).wait()

        @plsc.parallel_loop(
            0, _NCH, unroll=2,
            carry=tuple(jnp.full((16,), _NEG, jnp.float32)
                        for _ in range(_G)))
        def accs(i, acc):
            sl = pl.ds(i * 16, 16)
            return tuple(jnp.maximum(acc[g], inb[g, sl]) for g in range(_G))

        b0v = []
        for g in range(_G):
            sk, _ = plsc.sort_key_val(accs[g], iota, descending=True)
            b0v.append(jnp.full((16,), sk[_K - 1]))

        endv = [jnp.full((16,), (g + 1) * _CSTR - 1, jnp.int32)
                for g in range(_G)]

        @plsc.parallel_loop(
            0, _NCH, unroll=2,
            carry=tuple(jnp.full((16,), g * _CSTR, jnp.int32)
                        for g in range(_G)))
        def offs(i, off):
            sl = pl.ds(i * 16, 16)
            idxv = iota + i * 16
            new = []
            for g in range(_G):
                v = inb[g, sl]
                msk = v >= b0v[g]
                mi = msk.astype(jnp.int32)
                excl = plsc.cumsum(mi) - mi
                pos = jnp.minimum(off[g] + excl, endv[g])
                plsc.store_scatter(candi, [pos], idxv, mask=msk)
                new.append(off[g] + plsc.all_reduce_population_count(msk))
            return tuple(new)

        for g in range(_G):
            selbuf[pl.ds(g * 16, 16)] = jnp.full((16,), -1, jnp.int32)
            ncand = offs[g][0] - g * _CSTR
            cvec = jnp.full((16,), ncand, jnp.int32)

            def merge_step(vals, idxs, run):
                sa, _ = plsc.sort_key_val(vals, idxs, descending=False)
                mx = jnp.maximum(run, sa)
                rd, _ = plsc.sort_key_val(mx, iota, descending=True)
                return rd

            def select_plan(run):
                thr = run[_K - 1]
                rowmax = run[0]
                ne = _K - plsc.all_reduce_population_count(run > thr)
                return thr, rowmax, ne

            def select_step(vals, idxs, thrv, nev, eqc, soff):
                gt = vals > thrv
                eq = vals == thrv
                eqi = eq.astype(jnp.int32)
                excl = plsc.cumsum(eqi) - eqi
                sel = jnp.logical_or(
                    gt, jnp.logical_and(eq, (excl + eqc) < nev))
                seli = sel.astype(jnp.int32)
                sx = plsc.cumsum(seli) - seli
                plsc.store_scatter(selbuf, [soff + sx], idxs, mask=sel)
                return (eqc + plsc.all_reduce_population_count(eq),
                        soff + plsc.all_reduce_population_count(sel))

            @pl.when(ncand <= _CAP)
            def _(g=g, ncand=ncand, cvec=cvec):
                base = g * _CSTR
                candi[pl.ds(base + ncand, 16)] = zerosiv
                nch = (ncand + 15) // 16

                def body_c(j, run):
                    idxs = candi[pl.ds(base + j * 16, 16)]
                    vals = plsc.load_gather(inb, [jnp.full((16,), g, jnp.int32), idxs])
                    vals = jnp.where(iota + j * 16 < cvec, vals, -1.0)
                    return merge_step(vals, idxs, run)
                run = lax.fori_loop(0, nch, body_c,
                                    jnp.full((16,), _NEG, jnp.float32))
                thr, rowmax, ne = select_plan(run)
                thrv = jnp.full((16,), thr)

                @pl.when(rowmax > _EPS)
                def _():
                    def body_s(j, c):
                        idxs = candi[pl.ds(base + j * 16, 16)]
                        vals = plsc.load_gather(inb, [jnp.full((16,), g, jnp.int32), idxs])
                        vals = jnp.where(iota + j * 16 < cvec, vals, -1.0)
                        return select_step(vals, idxs, thrv, ne, *c)
                    lax.fori_loop(0, nch, body_s,
                                  (zerosiv, jnp.full((16,), g * 16,
                                                     jnp.int32)))

            @pl.when(ncand > _CAP)
            def _(g=g):
                def body_c(j, run):
                    sl = pl.ds(j * 16, 16)
                    return merge_step(inb[g, sl], iota + j * 16, run)
                run = lax.fori_loop(0, _NCH, body_c,
                                    jnp.full((16,), _NEG, jnp.float32))
                thr, rowmax, ne = select_plan(run)
                thrv = jnp.full((16,), thr)

                @pl.when(rowmax > _EPS)
                def _():
                    def body_s(j, c):
                        sl = pl.ds(j * 16, 16)
                        return select_step(inb[g, sl], iota + j * 16,
                                           thrv, ne, *c)
                    lax.fori_loop(0, _NCH, body_s,
                                  (zerosiv, jnp.full((16,), g * 16,
                                                     jnp.int32)))

        @plsc.parallel_loop(0, _NCH, unroll=2)
        def _(i):
            sl = pl.ds(i * 16, 16)
            for g in range(_G):
                inb[g, sl] = zerosv

        for g in range(_G):
            selv = selbuf[pl.ds(g * 16, 16)]
            m = selv >= 0
            gv = jnp.full((16,), g, jnp.int32)
            plsc.store_scatter(inb, [gv, selv], onesv, mask=m)
        pltpu.async_copy(inb, o_hbm.at[pl.ds(8 * t, 8), :], dsem).wait()
        return carry

    lax.fori_loop(0, _TPW, lambda k, c: tile_row(wid * _TPW + k, c),
                  jnp.zeros((), jnp.int32))


_mesh = plsc.VectorSubcoreMesh(core_axis_name="c", subcore_axis_name="s",
                               num_cores=_NC, num_subcores=_NS)

_topk_mask = functools.partial(
    pl.kernel,
    out_type=jax.ShapeDtypeStruct((_B * _N, _L), jnp.float32),
    mesh=_mesh,
    scratch_types=(
        pltpu.VMEM((_G, _L), jnp.float32),
        pltpu.VMEM((_G * _CSTR,), jnp.int32),
        pltpu.VMEM((_G * 16,), jnp.int32),
        pltpu.SemaphoreType.DMA,
    ),
    compiler_params=pltpu.CompilerParams(needs_layout_passes=False),
)(_sc_body)


@jax.jit
def kernel(metrics):
    b, n, l = metrics.shape
    out = _topk_mask(metrics.reshape(b * n, l))
    return out.reshape(b, n, l)


# speedup vs baseline: 1.0861x; 1.0007x over previous
import functools

import jax
import jax.numpy as jnp
from jax import lax
from jax.experimental import pallas as pl
from jax.experimental.pallas import tpu as pltpu
from jax.experimental.pallas import tpu_sc as plsc

_K = 13
_EPS = 1e-9
_B, _N, _L = 8, 64, 8400
_NCH = _L // 16
_NC, _NS = 2, 16
_NW = _NC * _NS
_TRS = _B * _N // 8
_TPW = _TRS // _NW
_G = 8
_CAP = 1008
_CSTR = 1024
_NEG = -3.0e38


def _sc_body(m_hbm, o_hbm, inb, candi, selbuf, dsem):
    wid = lax.axis_index("s") * _NC + lax.axis_index("c")
    iota = lax.iota(jnp.int32, 16)
    onesv = jnp.full((16,), 1.0, jnp.float32)
    zerosv = jnp.zeros((16,), jnp.float32)
    zerosiv = jnp.zeros((16,), jnp.int32)

    def tile_row(t, carry):
        pltpu.async_copy(m_hbm.at[pl.ds(8 * t, 8), :], inb, dsem).wait()

        @plsc.parallel_loop(
            0, _NCH, unroll=2,
            carry=tuple(jnp.full((16,), _NEG, jnp.float32)
                        for _ in range(_G)))
        def accs(i, acc):
            sl = pl.ds(i * 16, 16)
            return tuple(jnp.maximum(acc[g], inb[g, sl]) for g in range(_G))

        b0v = []
        for g in range(_G):
            sk, _ = plsc.sort_key_val(accs[g], iota, descending=True)
            b0v.append(jnp.full((16,), sk[_K - 1]))

        endv = [jnp.full((16,), (g + 1) * _CSTR - 1, jnp.int32)
                for g in range(_G)]

        @plsc.parallel_loop(
            0, _NCH, unroll=2,
            carry=tuple(jnp.full((16,), g * _CSTR, jnp.int32)
                        for g in range(_G)))
        def offs(i, off):
            sl = pl.ds(i * 16, 16)
            idxv = iota + i * 16
            new = []
            for g in range(_G):
                v = inb[g, sl]
                msk = v >= b0v[g]
                mi = msk.astype(jnp.int32)
                excl = plsc.cumsum(mi) - mi
                pos = jnp.minimum(off[g] + excl, endv[g])
                plsc.store_scatter(candi, [pos], idxv, mask=msk)
                new.append(off[g] + plsc.all_reduce_population_count(msk))
            return tuple(new)

        for g in range(_G):
            selbuf[pl.ds(g * 16, 16)] = jnp.full((16,), -1, jnp.int32)
            ncand = offs[g][0] - g * _CSTR
            cvec = jnp.full((16,), ncand, jnp.int32)

            def merge_step(vals, idxs, run):
                sa, _ = plsc.sort_key_val(vals, idxs, descending=False)
                mx = jnp.maximum(run, sa)
                rd, _ = plsc.sort_key_val(mx, iota, descending=True)
                return rd

            def select_plan(run):
                thr = run[_K - 1]
                rowmax = run[0]
                ne = _K - plsc.all_reduce_population_count(run > thr)
                return thr, rowmax, ne

            def select_step(vals, idxs, thrv, nev, eqc, soff):
                gt = vals > thrv
                eq = vals == thrv
                eqi = eq.astype(jnp.int32)
                excl = plsc.cumsum(eqi) - eqi
                sel = jnp.logical_or(
                    gt, jnp.logical_and(eq, (excl + eqc) < nev))
                seli = sel.astype(jnp.int32)
                sx = plsc.cumsum(seli) - seli
                plsc.store_scatter(selbuf, [soff + sx], idxs, mask=sel)
                return (eqc + plsc.all_reduce_population_count(eq),
                        soff + plsc.all_reduce_population_count(sel))

            @pl.when(ncand <= _CAP)
            def _(g=g, ncand=ncand, cvec=cvec):
                base = g * _CSTR
                candi[pl.ds(base + ncand, 16)] = zerosiv
                nch = (ncand + 15) // 16

                def body_c(j, run):
                    idxs = candi[pl.ds(base + j * 16, 16)]
                    vals = plsc.load_gather(inb, [jnp.full((16,), g, jnp.int32), idxs])
                    vals = jnp.where(iota + j * 16 < cvec, vals, -1.0)
                    return merge_step(vals, idxs, run)
                run = lax.fori_loop(0, nch, body_c,
                                    jnp.full((16,), _NEG, jnp.float32))
                thr, rowmax, ne = select_plan(run)
                thrv = jnp.full((16,), thr)

                @pl.when(rowmax > _EPS)
                def _():
                    def body_s(j, c):
                        idxs = candi[pl.ds(base + j * 16, 16)]
                        vals = plsc.load_gather(inb, [jnp.full((16,), g, jnp.int32), idxs])
                        vals = jnp.where(iota + j * 16 < cvec, vals, -1.0)
                        return select_step(vals, idxs, thrv, ne, *c)
                    lax.fori_loop(0, nch, body_s,
                                  (zerosiv, jnp.full((16,), g * 16,
                                                     jnp.int32)))

            @pl.when(ncand > _CAP)
            def _(g=g):
                def body_c(j, run):
                    sl = pl.ds(j * 16, 16)
                    return merge_step(inb[g, sl], iota + j * 16, run)
                run = lax.fori_loop(0, _NCH, body_c,
                                    jnp.full((16,), _NEG, jnp.float32))
                thr, rowmax, ne = select_plan(run)
                thrv = jnp.full((16,), thr)

                @pl.when(rowmax > _EPS)
                def _():
                    def body_s(j, c):
                        sl = pl.ds(j * 16, 16)
                        return select_step(inb[g, sl], iota + j * 16,
                                           thrv, ne, *c)
                    lax.fori_loop(0, _NCH, body_s,
                                  (zerosiv, jnp.full((16,), g * 16,
                                                     jnp.int32)))

        @plsc.parallel_loop(0, _NCH, unroll=2)
        def _(i):
            sl = pl.ds(i * 16, 16)
            for g in range(_G):
                inb[g, sl] = zerosv

        for g in range(_G):
            selv = selbuf[pl.ds(g * 16, 16)]
            m = selv >= 0
            gv = jnp.full((16,), g, jnp.int32)
            plsc.store_scatter(inb, [gv, selv], onesv, mask=m)
        pltpu.async_copy(inb, o_hbm.at[pl.ds(8 * t, 8), :], dsem).wait()
        return carry

    lax.fori_loop(0, _TPW, lambda k, c: tile_row(wid * _TPW + k, c),
                  jnp.zeros((), jnp.int32))


_mesh = plsc.VectorSubcoreMesh(core_axis_name="c", subcore_axis_name="s",
                               num_cores=_NC, num_subcores=_NS)

_topk_mask = functools.partial(
    pl.kernel,
    out_type=jax.ShapeDtypeStruct((_B * _N, _L), jnp.float32),
    mesh=_mesh,
    scratch_types=(
        pltpu.VMEM((_G, _L), jnp.float32),
        pltpu.VMEM((_G * _CSTR,), jnp.int32),
        pltpu.VMEM((_G * 16,), jnp.int32),
        pltpu.SemaphoreType.DMA,
    ),
    compiler_params=pltpu.CompilerParams(needs_layout_passes=False),
)(_sc_body)


@jax.jit
def kernel(metrics):
    b, n, l = metrics.shape
    out = _topk_mask(metrics.reshape(b * n, l))
    return out.reshape(b, n, l)
